# baseline (speedup 1.0000x reference)
import functools

import jax
import jax.numpy as jnp
from jax import lax
from jax.experimental import pallas as pl
from jax.experimental.pallas import tpu as pltpu
from jax.experimental.pallas import tpu_sc as plsc

B = 16384
M = 1000000
D = 64
Q = 1 << 18
NBLK = 8192
G1 = Q // NBLK
LAST_BLK = (M - 1) // NBLK

_INFO = plsc.get_sparse_core_info()
_NC = _INFO.num_cores
_NS = _INFO.num_subcores
_NW = _NC * _NS
_ROWS_PER_W = B // _NW
_CHUNK = 128
_NCHUNK = _ROWS_PER_W // _CHUNK


def _pack_words(t_blk, wp):
    lo = lax.dot_general(
        t_blk, wp[:, : D // 2], (((0,), (0,)), ((), ())),
        preferred_element_type=jnp.float32,
    ).astype(jnp.bfloat16)
    hi = lax.dot_general(
        t_blk, wp[:, D // 2 :], (((0,), (0,)), ((), ())),
        preferred_element_type=jnp.float32,
    ).astype(jnp.bfloat16)
    lo32 = lax.bitcast_convert_type(lo, jnp.uint16).astype(jnp.uint32)
    hi32 = lax.bitcast_convert_type(hi, jnp.uint16).astype(jnp.uint32)
    return lo32 | (hi32 << 16)


def _p4_body(t0_ref, t1_ref, t2_ref, t3_ref, wp_ref, out_ref):
    wp = wp_ref[...]
    out_ref[:, : 32] = _pack_words(t0_ref[...], wp)
    out_ref[:, 32: 64] = _pack_words(t1_ref[...], wp)
    out_ref[:, 64: 96] = _pack_words(t2_ref[...], wp)
    out_ref[:, 96:128] = _pack_words(t3_ref[...], wp)


def _transform(tT, Wp):
    def _tspec(q):
        return pl.BlockSpec(
            (D, NBLK), lambda g, q=q: (0, jnp.minimum(g + q * G1, LAST_BLK))
        )

    return pl.pallas_call(
        _p4_body,
        grid=(G1,),
        in_specs=[
            _tspec(0), _tspec(1), _tspec(2), _tspec(3),
            pl.BlockSpec((D, D), lambda g: (0, 0)),
        ],
        out_specs=pl.BlockSpec((NBLK, 2 * D), lambda g: (g, 0)),
        out_shape=jax.ShapeDtypeStruct((Q, 2 * D), jnp.uint32),
        compiler_params=pltpu.CompilerParams(
            dimension_semantics=("parallel",),
        ),
    )(tT, tT, tT, tT, Wp)


def _make_sc_gather():
    mesh = plsc.VectorSubcoreMesh(core_axis_name="c", subcore_axis_name="s")

    @functools.partial(
        pl.kernel,
        mesh=mesh,
        out_type=jax.ShapeDtypeStruct((B, 2 * D), jnp.uint32),
        scratch_types=[
            pltpu.VMEM((_ROWS_PER_W,), jnp.int32),
            pltpu.VMEM((_ROWS_PER_W, 2 * D), jnp.uint32),
            pltpu.SemaphoreType.DMA,
        ],
    )
    def gather_kernel(p4_hbm, idx_hbm, g4_hbm, idx_v, rows_v, sem):
        wid = lax.axis_index("s") * _NC + lax.axis_index("c")
        base = wid * _ROWS_PER_W
        pltpu.sync_copy(idx_hbm.at[pl.ds(base, _ROWS_PER_W)], idx_v)

        def fold(g, _):
            vec = idx_v[pl.ds(g * 16, 16)]
            idx_v[pl.ds(g * 16, 16)] = jnp.bitwise_and(vec, Q - 1)
            return 0

        lax.fori_loop(0, _ROWS_PER_W // 16, fold, 0)

        copies = [
            pltpu.make_async_copy(
                p4_hbm.at[idx_v.at[pl.ds(c * _CHUNK, _CHUNK)]],
                rows_v.at[pl.ds(c * _CHUNK, _CHUNK)],
                sem,
            )
            for c in range(_NCHUNK)
        ]
        for cp in copies:
            cp.start()
        for cp in copies:
            cp.wait()
        pltpu.sync_copy(rows_v, g4_hbm.at[pl.ds(base, _ROWS_PER_W)])

    return gather_kernel


_sc_gather = _make_sc_gather()


_BLK = 2048


def _merge_body(old_ref, g4_ref, idx_ref, w1p_ref, bp_ref, out_ref):
    g4 = g4_ref[...]
    i = idx_ref[...]
    w01 = jnp.where(i < Q, g4[:, :32], g4[:, 32:64])
    w23 = jnp.where(i < 3 * Q, g4[:, 64:96], g4[:, 96:128])
    gw = lax.bitcast_convert_type(
        jnp.where(i < 2 * Q, w01, w23), jnp.int32
    )
    f_even = lax.bitcast_convert_type(lax.shift_left(gw, 16), jnp.float32)
    f_odd = lax.bitcast_convert_type(
        lax.bitwise_and(gw, jnp.int32(-65536)), jnp.float32
    )
    out_ref[...] = (
        lax.dot_general(
            old_ref[...], w1p_ref[...], (((1,), (0,)), ((), ())),
            preferred_element_type=jnp.float32,
        )
        + jnp.concatenate([f_even, f_odd], axis=1)
        + bp_ref[...]
    )


def _merge(old_mems, g4, idxc, W1p, bp):
    return pl.pallas_call(
        _merge_body,
        grid=(B // _BLK,),
        in_specs=[
            pl.BlockSpec((_BLK, D), lambda i: (i, 0)),
            pl.BlockSpec((_BLK, 2 * D), lambda i: (i, 0)),
            pl.BlockSpec((_BLK, 1), lambda i: (i, 0)),
            pl.BlockSpec((D, D), lambda i: (0, 0)),
            pl.BlockSpec((1, D), lambda i: (0, 0)),
        ],
        out_specs=pl.BlockSpec((_BLK, D), lambda i: (i, 0)),
        out_shape=jax.ShapeDtypeStruct((B, D), jnp.float32),
        compiler_params=pltpu.CompilerParams(
            dimension_semantics=("parallel",),
        ),
    )(old_mems, g4, idxc, W1p, bp)


def kernel(old_mems, new_mems, indices, W, b):
    idx = indices.astype(jnp.int32)
    wperm = jnp.concatenate([W[:, 0::2], W[:, 1::2]], axis=1)
    Wp = wperm[D:, :]
    W1p = wperm[:D, :]
    bp = jnp.concatenate([b[0::2], b[1::2]]).reshape(1, D)
    p4 = _transform(new_mems.T, Wp)
    g4 = _sc_gather(p4, idx)
    out2 = _merge(old_mems, g4, idx.reshape(B, 1), W1p, bp)
    return jnp.stack(
        [out2[:, : D // 2], out2[:, D // 2 :]], axis=-1
    ).reshape(B, D)


# speedup vs baseline: 1.2394x; 1.2394x over previous
import functools

import jax
import jax.numpy as jnp
from jax import lax
from jax.experimental import pallas as pl
from jax.experimental.pallas import tpu as pltpu
from jax.experimental.pallas import tpu_sc as plsc

B = 16384
M = 1000000
D = 64
H = 524288
NBLK = 16384
G1 = H // NBLK
LAST_BLK = (M - 1) // NBLK

_INFO = plsc.get_sparse_core_info()
_NC = _INFO.num_cores
_NS = _INFO.num_subcores
_NW = _NC * _NS
_ROWS_PER_W = B // _NW
_CHUNK = 128
_NCHUNK = _ROWS_PER_W // _CHUNK


def _p2_body(t1_ref, t2_ref, w_ref, out_ref):
    w2 = w_ref[D:, :]
    out_ref[:, :D] = lax.dot_general(
        t1_ref[...], w2, (((0,), (0,)), ((), ())),
        preferred_element_type=jnp.float32,
    )
    out_ref[:, D:] = lax.dot_general(
        t2_ref[...], w2, (((0,), (0,)), ((), ())),
        preferred_element_type=jnp.float32,
    )


def _transform(tT, W):
    return pl.pallas_call(
        _p2_body,
        grid=(G1,),
        in_specs=[
            pl.BlockSpec((D, NBLK), lambda g: (0, g)),
            pl.BlockSpec(
                (D, NBLK), lambda g: (0, jnp.minimum(g + G1, LAST_BLK))
            ),
            pl.BlockSpec((2 * D, D), lambda g: (0, 0)),
        ],
        out_specs=pl.BlockSpec((NBLK, 2 * D), lambda g: (g, 0)),
        out_shape=jax.ShapeDtypeStruct((H, 2 * D), jnp.float32),
        compiler_params=pltpu.CompilerParams(
            dimension_semantics=("parallel",),
        ),
    )(tT, tT, W)


def _make_sc_gather():
    mesh = plsc.VectorSubcoreMesh(core_axis_name="c", subcore_axis_name="s")

    @functools.partial(
        pl.kernel,
        mesh=mesh,
        out_type=jax.ShapeDtypeStruct((B, 2 * D), jnp.float32),
        scratch_types=[
            pltpu.VMEM((_ROWS_PER_W,), jnp.int32),
            pltpu.VMEM((_ROWS_PER_W, 2 * D), jnp.float32),
            pltpu.SemaphoreType.DMA,
        ],
    )
    def gather_kernel(p2_hbm, idx_hbm, g2_hbm, idx_v, rows_v, sem):
        wid = lax.axis_index("s") * _NC + lax.axis_index("c")
        base = wid * _ROWS_PER_W
        pltpu.sync_copy(idx_hbm.at[pl.ds(base, _ROWS_PER_W)], idx_v)

        def fold(g, _):
            vec = idx_v[pl.ds(g * 16, 16)]
            idx_v[pl.ds(g * 16, 16)] = jnp.where(vec >= H, vec - H, vec)
            return 0

        lax.fori_loop(0, _ROWS_PER_W // 16, fold, 0)

        copies = [
            pltpu.make_async_copy(
                p2_hbm.at[idx_v.at[pl.ds(c * _CHUNK, _CHUNK)]],
                rows_v.at[pl.ds(c * _CHUNK, _CHUNK)],
                sem,
            )
            for c in range(_NCHUNK)
        ]
        for cp in copies:
            cp.start()
        for cp in copies:
            cp.wait()
        pltpu.sync_copy(rows_v, g2_hbm.at[pl.ds(base, _ROWS_PER_W)])

    return gather_kernel


_sc_gather = _make_sc_gather()


_BLK = 2048


def _merge_body(old_ref, g2_ref, idx_ref, w_ref, b_ref, out_ref):
    g2 = g2_ref[...]
    sel = idx_ref[...] >= H
    g = jnp.where(sel, g2[:, D:], g2[:, :D])
    out_ref[...] = (
        lax.dot_general(
            old_ref[...], w_ref[:D, :], (((1,), (0,)), ((), ())),
            preferred_element_type=jnp.float32,
        )
        + g
        + b_ref[...]
    )


def _merge(old_mems, g2, idxc, W, b2d):
    return pl.pallas_call(
        _merge_body,
        grid=(B // _BLK,),
        in_specs=[
            pl.BlockSpec((_BLK, D), lambda i: (i, 0)),
            pl.BlockSpec((_BLK, 2 * D), lambda i: (i, 0)),
            pl.BlockSpec((_BLK, 1), lambda i: (i, 0)),
            pl.BlockSpec((2 * D, D), lambda i: (0, 0)),
            pl.BlockSpec((1, D), lambda i: (0, 0)),
        ],
        out_specs=pl.BlockSpec((_BLK, D), lambda i: (i, 0)),
        out_shape=jax.ShapeDtypeStruct((B, D), jnp.float32),
        compiler_params=pltpu.CompilerParams(
            dimension_semantics=("parallel",),
        ),
    )(old_mems, g2, idxc, W, b2d)


def kernel(old_mems, new_mems, indices, W, b):
    idx = indices.astype(jnp.int32)
    p2 = _transform(new_mems.T, W)
    g2 = _sc_gather(p2, idx)
    return _merge(old_mems, g2, idx.reshape(B, 1), W, b.reshape(1, D))
